# baseline (speedup 1.0000x reference)
import functools

import jax
import jax.numpy as jnp
from jax import lax
from jax.experimental import pallas as pl
from jax.experimental.pallas import tpu as pltpu
from jax.experimental.pallas import tpu_sc as plsc

_ROWS = 16384
_COLS = 4096
_PERIOD = 4096

_NW = 32
_ROWS_PER_W = _ROWS // _NW
_CHUNK = 8
_NCH = _ROWS_PER_W // _CHUNK
_NBUF = 3

_mesh = plsc.VectorSubcoreMesh(core_axis_name="c", subcore_axis_name="s")


@functools.partial(
    pl.kernel,
    mesh=_mesh,
    out_type=jax.ShapeDtypeStruct((_ROWS, _COLS), jnp.float32),
    scratch_types=(
        [pltpu.VMEM((_CHUNK, _COLS), jnp.float32) for _ in range(_NBUF)]
        + [pltpu.SemaphoreType.DMA for _ in range(2 * _NBUF)]
    ),
)
def _sc_copy(x_hbm, o_hbm, *scratch):
    bufs = scratch[:_NBUF]
    isems = scratch[_NBUF:2 * _NBUF]
    osems = scratch[2 * _NBUF:]
    wid = lax.axis_index("s") * 2 + lax.axis_index("c")
    base = wid * _ROWS_PER_W

    def in_copy(i):
        b = i % _NBUF
        return pltpu.make_async_copy(
            x_hbm.at[pl.ds(base + i * _CHUNK, _CHUNK)],
            bufs[b],
            isems[b],
        )

    def out_copy(i):
        b = i % _NBUF
        return pltpu.make_async_copy(
            bufs[b],
            o_hbm.at[pl.ds(base + i * _CHUNK, _CHUNK)],
            osems[b],
        )

    for s in range(_NBUF):
        in_copy(s).start()
    for i in range(_NCH):
        oldest = i - (_NBUF - 1)
        if oldest >= 0 and oldest + _NBUF < _NCH:
            out_copy(oldest).wait()
            in_copy(oldest + _NBUF).start()
        in_copy(i).wait()
        out_copy(i).start()
    for i in range(max(_NCH - _NBUF, 0), _NCH):
        out_copy(i).wait()


def kernel(x):
    out = _sc_copy(x)
    return jnp.reshape(out, (_ROWS // _PERIOD, _PERIOD, _COLS))


# speedup vs baseline: 1.0083x; 1.0083x over previous
import functools

import jax
import jax.numpy as jnp
from jax import lax
from jax.experimental import pallas as pl
from jax.experimental.pallas import tpu as pltpu
from jax.experimental.pallas import tpu_sc as plsc

_ROWS = 16384
_COLS = 4096
_PERIOD = 4096

_NW = 32
_ROWS_PER_W = _ROWS // _NW
_CHUNK = 4
_NCH = _ROWS_PER_W // _CHUNK
_NBUF = 6

_mesh = plsc.VectorSubcoreMesh(core_axis_name="c", subcore_axis_name="s")


@functools.partial(
    pl.kernel,
    mesh=_mesh,
    out_type=jax.ShapeDtypeStruct((_ROWS, _COLS), jnp.float32),
    scratch_types=(
        [pltpu.VMEM((_CHUNK, _COLS), jnp.float32) for _ in range(_NBUF)]
        + [pltpu.SemaphoreType.DMA for _ in range(2 * _NBUF)]
    ),
)
def _sc_copy(x_hbm, o_hbm, *scratch):
    bufs = scratch[:_NBUF]
    isems = scratch[_NBUF:2 * _NBUF]
    osems = scratch[2 * _NBUF:]
    wid = lax.axis_index("s") * 2 + lax.axis_index("c")
    base = wid * _ROWS_PER_W

    def in_copy(i):
        b = i % _NBUF
        return pltpu.make_async_copy(
            x_hbm.at[pl.ds(base + i * _CHUNK, _CHUNK)],
            bufs[b],
            isems[b],
        )

    def out_copy(i):
        b = i % _NBUF
        return pltpu.make_async_copy(
            bufs[b],
            o_hbm.at[pl.ds(base + i * _CHUNK, _CHUNK)],
            osems[b],
        )

    for s in range(_NBUF):
        in_copy(s).start()
    for i in range(_NCH):
        oldest = i - (_NBUF - 1)
        if oldest >= 0 and oldest + _NBUF < _NCH:
            out_copy(oldest).wait()
            in_copy(oldest + _NBUF).start()
        in_copy(i).wait()
        out_copy(i).start()
    for i in range(max(_NCH - _NBUF, 0), _NCH):
        out_copy(i).wait()


def kernel(x):
    out = _sc_copy(x)
    return jnp.reshape(out, (_ROWS // _PERIOD, _PERIOD, _COLS))


# speedup vs baseline: 1.0254x; 1.0169x over previous
import functools

import jax
import jax.numpy as jnp
from jax import lax
from jax.experimental import pallas as pl
from jax.experimental.pallas import tpu as pltpu
from jax.experimental.pallas import tpu_sc as plsc

_ROWS = 16384
_COLS = 4096
_PERIOD = 4096

_NW = 32
_NSUB = 16
_ROWS_PER_W = _ROWS // _NW

_TCHUNK = 8
_SCHUNK = 4
_NT = 40
_NS = 48
_BT = 3
_BS = 2

_mesh = plsc.VectorSubcoreMesh(core_axis_name="c", subcore_axis_name="s")


@functools.partial(
    pl.kernel,
    mesh=_mesh,
    out_type=jax.ShapeDtypeStruct((_ROWS, _COLS), jnp.float32),
    scratch_types=(
        [pltpu.VMEM((_TCHUNK, _COLS), jnp.float32) for _ in range(_BT)]
        + [pltpu.VMEM_SHARED((_NSUB, _BS, _SCHUNK, _COLS), jnp.float32)]
        + [pltpu.SemaphoreType.DMA for _ in range(2 * (_BT + _BS))]
    ),
)
def _sc_copy(x_hbm, o_hbm, *scratch):
    bufs = scratch[:_BT]
    shared = scratch[_BT]
    sems = scratch[_BT + 1:]
    t_isems = sems[:_BT]
    t_osems = sems[_BT:2 * _BT]
    s_isems = sems[2 * _BT:2 * _BT + _BS]
    s_osems = sems[2 * _BT + _BS:]

    sid = lax.axis_index("s")
    wid = sid * 2 + lax.axis_index("c")
    base = wid * _ROWS_PER_W
    sbase = base + _NT * _TCHUNK

    def t_in(i):
        b = i % _BT
        return pltpu.make_async_copy(
            x_hbm.at[pl.ds(base + i * _TCHUNK, _TCHUNK)], bufs[b], t_isems[b])

    def t_out(i):
        b = i % _BT
        return pltpu.make_async_copy(
            bufs[b], o_hbm.at[pl.ds(base + i * _TCHUNK, _TCHUNK)], t_osems[b])

    def s_in(i):
        b = i % _BS
        return pltpu.make_async_copy(
            x_hbm.at[pl.ds(sbase + i * _SCHUNK, _SCHUNK)],
            shared.at[sid, b], s_isems[b])

    def s_out(i):
        b = i % _BS
        return pltpu.make_async_copy(
            shared.at[sid, b],
            o_hbm.at[pl.ds(sbase + i * _SCHUNK, _SCHUNK)], s_osems[b])

    for b in range(_BT):
        t_in(b).start()
    for b in range(_BS):
        s_in(b).start()

    def s_step(j):
        s_oldest = j - (_BS - 1)
        if s_oldest >= 0 and s_oldest + _BS < _NS:
            s_out(s_oldest).wait()
            s_in(s_oldest + _BS).start()
        s_in(j).wait()
        s_out(j).start()

    s_i = 0
    for i in range(_NT):
        oldest = i - (_BT - 1)
        if oldest >= 0 and oldest + _BT < _NT:
            t_out(oldest).wait()
            t_in(oldest + _BT).start()
        t_in(i).wait()
        t_out(i).start()
        while s_i < (i + 1) * _NS // _NT:
            s_step(s_i)
            s_i += 1
    while s_i < _NS:
        s_step(s_i)
        s_i += 1

    for i in range(max(_NT - _BT, 0), _NT):
        t_out(i).wait()
    for j in range(max(_NS - _BS, 0), _NS):
        s_out(j).wait()


def kernel(x):
    out = _sc_copy(x)
    return jnp.reshape(out, (_ROWS // _PERIOD, _PERIOD, _COLS))
